# baseline (speedup 1.0000x reference)
import functools
import jax
import jax.numpy as jnp
from jax import lax
from jax.experimental import pallas as pl
from jax.experimental.pallas import tpu as pltpu, tpu_sc as plsc

_NC, _NS, _LANES = 2, 16, 16
_NW = _NC * _NS


def _make_sc_add(B, S, D):
    SW = S // _NW
    CH = 16
    NCHUNK = SW // CH
    NSLICE = D // _LANES
    assert NCHUNK % 2 == 0 and NCHUNK >= 4

    def _add_loop(xv, pev):
        def add_body(r, _):
            for j in range(NSLICE):
                sl = pl.ds(j * _LANES, _LANES)
                xv[r, sl] = xv[r, sl] + pev[r, sl]
            return 0

        lax.fori_loop(0, CH, add_body, 0)

    def body(x_hbm, pe_hbm, o_hbm, *scratch):
        xbufs = scratch[0:B]
        pebufs = scratch[B:B + 2]
        in_sems = scratch[B + 2:2 * B + 2]
        out_sems = scratch[2 * B + 2:3 * B + 2]
        pe_sems = scratch[3 * B + 2:3 * B + 4]

        wid = lax.axis_index("s") * _NC + lax.axis_index("c")
        base = wid * SW

        def start_in(c, b):
            row = b * S + base + c * CH
            pltpu.async_copy(x_hbm.at[pl.ds(row, CH), :], xbufs[b],
                             in_sems[b])

        def wait_in(b):
            pltpu.make_async_copy(x_hbm.at[pl.ds(base, CH), :], xbufs[b],
                                  in_sems[b]).wait()

        def start_out(c, b):
            row = b * S + base + c * CH
            pltpu.async_copy(xbufs[b], o_hbm.at[pl.ds(row, CH), :],
                             out_sems[b])

        def wait_out(b):
            pltpu.make_async_copy(xbufs[b], o_hbm.at[pl.ds(base, CH), :],
                                  out_sems[b]).wait()

        def start_pe(c, parity):
            pltpu.async_copy(pe_hbm.at[pl.ds(base + c * CH, CH), :],
                             pebufs[parity], pe_sems[parity])

        def wait_pe(parity):
            pltpu.make_async_copy(pe_hbm.at[pl.ds(base, CH), :],
                                  pebufs[parity], pe_sems[parity]).wait()

        def proc_chunk(c, parity, prefetch):
            if prefetch:
                start_pe(c + 1, 1 - parity)
            wait_pe(parity)
            for b in range(B):
                wait_in(b)
                _add_loop(xbufs[b], pebufs[parity])
                start_out(c, b)
            if prefetch:
                for b in range(B):
                    wait_out(b)
                    start_in(c + 1, b)

        start_pe(0, 0)
        for b in range(B):
            start_in(0, b)

        def pair_body(cc, _):
            c = cc * 2
            proc_chunk(c, 0, True)
            proc_chunk(c + 1, 1, True)
            return 0

        lax.fori_loop(0, NCHUNK // 2 - 1, pair_body, 0)

        proc_chunk(NCHUNK - 2, 0, True)
        proc_chunk(NCHUNK - 1, 1, False)
        for b in range(B):
            wait_out(b)

    mesh = plsc.VectorSubcoreMesh(core_axis_name="c", subcore_axis_name="s")
    return pl.kernel(
        body,
        out_type=jax.ShapeDtypeStruct((B * S, D), jnp.float32),
        mesh=mesh,
        scratch_types=(
            [pltpu.VMEM((CH, D), jnp.float32) for _ in range(B)]
            + [pltpu.VMEM((CH, D), jnp.float32) for _ in range(2)]
            + [pltpu.SemaphoreType.DMA for _ in range(B)]
            + [pltpu.SemaphoreType.DMA for _ in range(B)]
            + [pltpu.SemaphoreType.DMA for _ in range(2)]
        ),
    )


def kernel(x, pe):
    B, S, D = x.shape
    sc_add = _make_sc_add(B, S, D)
    out = sc_add(x.reshape(B * S, D), pe)
    return out.reshape(B, S, D)
